# Initial kernel scaffold; baseline (speedup 1.0000x reference)
import jax
import jax.numpy as jnp
from jax.experimental import pallas as pl


def kernel(x, batch_idx, W1, b1, g1, be1, W2, b2, g2, be2, W3, b3):
    raise NotImplementedError("write your pallas kernel here")


# speedup vs baseline: 3.2646x; 3.2646x over previous
import functools

import jax
import jax.numpy as jnp
from jax import lax
from jax.experimental import pallas as pl
from jax.experimental.pallas import tpu as pltpu
from jax.experimental.pallas import tpu_sc as plsc

_N, _D, _B = 50000, 512, 64
_H = _D // 16
_C = 80
_NCH = _N // _C
_NW = 32
_CHUNKS_LO = 19
_EXTRA = _NCH - _NW * _CHUNKS_LO


def _sc_pool(x, batch_idx):
    mesh = plsc.VectorSubcoreMesh(core_axis_name="c", subcore_axis_name="s")

    @functools.partial(
        pl.kernel,
        out_type=[
            jax.ShapeDtypeStruct((_NW, _B, _D), jnp.float32),
            jax.ShapeDtypeStruct((_NW, _B, _D), jnp.float32),
            jax.ShapeDtypeStruct((_NW, _B, 16), jnp.float32),
        ],
        mesh=mesh,
        scratch_types=[
            pltpu.VMEM((_C, _D), jnp.float32),
            pltpu.VMEM((_C,), jnp.int32),
            pltpu.VMEM((_B, _D), jnp.float32),
            pltpu.VMEM((_B, _D), jnp.float32),
            pltpu.VMEM((_B, 16), jnp.float32),
        ],
    )
    def k(x_hbm, idx_hbm, pmax_hbm, psum_hbm, pcnt_hbm, xbuf, idbuf, amax, asum, acnt):
        cid = lax.axis_index("c")
        sid = lax.axis_index("s")
        wid = sid * 2 + cid

        neg = jnp.full((16,), -jnp.inf, dtype=jnp.float32)
        zero = jnp.zeros((16,), dtype=jnp.float32)
        one = jnp.ones((16,), dtype=jnp.float32)

        def init_body(i, carry):
            b = i // _H
            h = i % _H
            amax[b, pl.ds(h * 16, 16)] = neg
            asum[b, pl.ds(h * 16, 16)] = zero
            return carry

        lax.fori_loop(0, _B * _H, init_body, 0)

        def init_cnt(b, carry):
            acnt[b, :] = zero
            return carry

        lax.fori_loop(0, _B, init_cnt, 0)

        nch = jnp.where(wid < _EXTRA, _CHUNKS_LO + 1, _CHUNKS_LO)
        start = wid * _CHUNKS_LO + jnp.minimum(wid, _EXTRA)

        def chunk_body(j, carry):
            c = start + j
            pltpu.sync_copy(x_hbm.at[pl.ds(c * _C, _C), :], xbuf)
            pltpu.sync_copy(idx_hbm.at[pl.ds(c * _C, _C)], idbuf)

            def group_body(g, gcarry):
                idvec = idbuf[pl.ds(g * 16, 16)]
                for l in range(16):
                    seg = idvec[l]
                    r = g * 16 + l
                    acnt[seg, :] = acnt[seg, :] + one
                    for h in range(_H):
                        sl = pl.ds(h * 16, 16)
                        v = xbuf[r, sl]
                        amax[seg, sl] = jnp.maximum(amax[seg, sl], v)
                        asum[seg, sl] = asum[seg, sl] + v
                return gcarry

            lax.fori_loop(0, _C // 16, group_body, 0)
            return carry

        lax.fori_loop(0, nch, chunk_body, 0)

        pltpu.sync_copy(amax, pmax_hbm.at[wid])
        pltpu.sync_copy(asum, psum_hbm.at[wid])
        pltpu.sync_copy(acnt, pcnt_hbm.at[wid])

    return k(x, batch_idx)


def _tc_head_body(pmax_ref, psum_ref, pcnt_ref,
                  w1_ref, b1_ref, g1_ref, be1_ref,
                  w2_ref, b2_ref, g2_ref, be2_ref,
                  w3_ref, b3_ref, out_ref):
    xmax = jnp.max(pmax_ref[...], axis=0)
    xsum = jnp.sum(psum_ref[...], axis=0)
    cnt = jnp.sum(pcnt_ref[...], axis=0)[:, :1]
    xmean = xsum / jnp.maximum(cnt, 1.0)
    xx = jnp.concatenate([xmax, xmean], axis=1)

    def bn_lrelu(h, g, b):
        m = jnp.mean(h, axis=0)
        v = jnp.mean((h - m) ** 2, axis=0)
        h = (h - m) * lax.rsqrt(v + 1e-5) * g + b
        return jnp.where(h > 0, h, 0.01 * h)

    h = jnp.dot(xx, w1_ref[...], preferred_element_type=jnp.float32) + b1_ref[...]
    h = bn_lrelu(h, g1_ref[...], be1_ref[...])
    h = jnp.dot(h, w2_ref[...], preferred_element_type=jnp.float32) + b2_ref[...]
    h = bn_lrelu(h, g2_ref[...], be2_ref[...])
    out_ref[...] = jnp.dot(h, w3_ref[...], preferred_element_type=jnp.float32) + b3_ref[...]


def kernel(x, batch_idx, W1, b1, g1, be1, W2, b2, g2, be2, W3, b3):
    pmax, psum, pcnt = _sc_pool(x, batch_idx)
    out = pl.pallas_call(
        _tc_head_body,
        out_shape=jax.ShapeDtypeStruct((_B, 8), jnp.float32),
    )(pmax, psum, pcnt, W1, b1, g1, be1, W2, b2, g2, be2, W3, b3)
    return out


# speedup vs baseline: 6.4403x; 1.9728x over previous
import functools

import jax
import jax.numpy as jnp
from jax import lax
from jax.experimental import pallas as pl
from jax.experimental.pallas import tpu as pltpu
from jax.experimental.pallas import tpu_sc as plsc

_N, _D, _B = 50000, 512, 64
_H = _D // 16
_C = 80
_NCH = _N // _C
_NW = 32
_CHUNKS_LO = 19
_EXTRA = _NCH - _NW * _CHUNKS_LO


def _sc_pool(x, batch_idx):
    mesh = plsc.VectorSubcoreMesh(core_axis_name="c", subcore_axis_name="s")

    @functools.partial(
        pl.kernel,
        out_type=[
            jax.ShapeDtypeStruct((_NW, _B, _D), jnp.float32),
            jax.ShapeDtypeStruct((_NW, _B, _D), jnp.float32),
            jax.ShapeDtypeStruct((_NW, _B, 16), jnp.float32),
        ],
        mesh=mesh,
        scratch_types=[
            pltpu.VMEM((_C, _D), jnp.float32),
            pltpu.VMEM((_C,), jnp.int32),
            pltpu.VMEM((_B, _D), jnp.float32),
            pltpu.VMEM((_B, _D), jnp.float32),
            pltpu.VMEM((_B, 16), jnp.float32),
        ],
    )
    def k(x_hbm, idx_hbm, pmax_hbm, psum_hbm, pcnt_hbm, xbuf, idbuf, amax, asum, acnt):
        cid = lax.axis_index("c")
        sid = lax.axis_index("s")
        wid = sid * 2 + cid

        neg = jnp.full((16,), -jnp.inf, dtype=jnp.float32)
        zero = jnp.zeros((16,), dtype=jnp.float32)
        one = jnp.ones((16,), dtype=jnp.float32)

        def init_body(i, carry):
            b = i // _H
            h = i % _H
            amax[b, pl.ds(h * 16, 16)] = neg
            asum[b, pl.ds(h * 16, 16)] = zero
            return carry

        lax.fori_loop(0, _B * _H, init_body, 0)

        def init_cnt(b, carry):
            acnt[b, :] = zero
            return carry

        lax.fori_loop(0, _B, init_cnt, 0)

        nch = jnp.where(wid < _EXTRA, _CHUNKS_LO + 1, _CHUNKS_LO)
        start = wid * _CHUNKS_LO + jnp.minimum(wid, _EXTRA)

        def chunk_body(j, carry):
            c = start + j
            pltpu.sync_copy(x_hbm.at[pl.ds(c * _C, _C), :], xbuf)
            pltpu.sync_copy(idx_hbm.at[pl.ds(c * _C, _C)], idbuf)

            def group_body(g, gcarry):
                idvec = idbuf[pl.ds(g * 16, 16)]
                seg0 = idvec[0]
                seg15 = idvec[15]

                @pl.when(seg0 == seg15)
                def _fast():
                    acnt[seg0, :] = acnt[seg0, :] + 16.0
                    for half in range(2):
                        accm = []
                        accs = []
                        for h in range(16):
                            sl = pl.ds((half * 16 + h) * 16, 16)
                            v = xbuf[g * 16, sl]
                            accm.append(v)
                            accs.append(v)
                        for l in range(1, 16):
                            r = g * 16 + l
                            for h in range(16):
                                sl = pl.ds((half * 16 + h) * 16, 16)
                                v = xbuf[r, sl]
                                accm[h] = jnp.maximum(accm[h], v)
                                accs[h] = accs[h] + v
                        for h in range(16):
                            sl = pl.ds((half * 16 + h) * 16, 16)
                            amax[seg0, sl] = jnp.maximum(amax[seg0, sl], accm[h])
                            asum[seg0, sl] = asum[seg0, sl] + accs[h]

                @pl.when(seg0 != seg15)
                def _slow():
                    for l in range(16):
                        seg = idvec[l]
                        r = g * 16 + l
                        acnt[seg, :] = acnt[seg, :] + one
                        for h in range(_H):
                            sl = pl.ds(h * 16, 16)
                            v = xbuf[r, sl]
                            amax[seg, sl] = jnp.maximum(amax[seg, sl], v)
                            asum[seg, sl] = asum[seg, sl] + v

                return gcarry

            lax.fori_loop(0, _C // 16, group_body, 0)
            return carry

        lax.fori_loop(0, nch, chunk_body, 0)

        pltpu.sync_copy(amax, pmax_hbm.at[wid])
        pltpu.sync_copy(asum, psum_hbm.at[wid])
        pltpu.sync_copy(acnt, pcnt_hbm.at[wid])

    return k(x, batch_idx)


def _tc_head_body(pmax_ref, psum_ref, pcnt_ref,
                  w1_ref, b1_ref, g1_ref, be1_ref,
                  w2_ref, b2_ref, g2_ref, be2_ref,
                  w3_ref, b3_ref, out_ref):
    xmax = jnp.max(pmax_ref[...], axis=0)
    xsum = jnp.sum(psum_ref[...], axis=0)
    cnt = jnp.sum(pcnt_ref[...], axis=0)[:, :1]
    xmean = xsum / jnp.maximum(cnt, 1.0)
    xx = jnp.concatenate([xmax, xmean], axis=1)

    def bn_lrelu(h, g, b):
        m = jnp.mean(h, axis=0)
        v = jnp.mean((h - m) ** 2, axis=0)
        h = (h - m) * lax.rsqrt(v + 1e-5) * g + b
        return jnp.where(h > 0, h, 0.01 * h)

    h = jnp.dot(xx, w1_ref[...], preferred_element_type=jnp.float32) + b1_ref[...]
    h = bn_lrelu(h, g1_ref[...], be1_ref[...])
    h = jnp.dot(h, w2_ref[...], preferred_element_type=jnp.float32) + b2_ref[...]
    h = bn_lrelu(h, g2_ref[...], be2_ref[...])
    out_ref[...] = jnp.dot(h, w3_ref[...], preferred_element_type=jnp.float32) + b3_ref[...]


def kernel(x, batch_idx, W1, b1, g1, be1, W2, b2, g2, be2, W3, b3):
    pmax, psum, pcnt = _sc_pool(x, batch_idx)
    out = pl.pallas_call(
        _tc_head_body,
        out_shape=jax.ShapeDtypeStruct((_B, 8), jnp.float32),
    )(pmax, psum, pcnt, W1, b1, g1, be1, W2, b2, g2, be2, W3, b3)
    return out


# speedup vs baseline: 9.6709x; 1.5016x over previous
import functools

import jax
import jax.numpy as jnp
from jax import lax
from jax.experimental import pallas as pl
from jax.experimental.pallas import tpu as pltpu
from jax.experimental.pallas import tpu_sc as plsc

_N, _D, _B = 50000, 512, 64
_H = _D // 16
_C = 80
_G = _C // 16
_NCH = _N // _C
_NW = 32
_CHUNKS_LO = 19
_EXTRA = _NCH - _NW * _CHUNKS_LO


def _tree_max(vs):
    while len(vs) > 1:
        vs = [jnp.maximum(vs[i], vs[i + 1]) for i in range(0, len(vs) - 1, 2)] + (
            [vs[-1]] if len(vs) % 2 else [])
    return vs[0]


def _tree_sum(vs):
    while len(vs) > 1:
        vs = [vs[i] + vs[i + 1] for i in range(0, len(vs) - 1, 2)] + (
            [vs[-1]] if len(vs) % 2 else [])
    return vs[0]


def _sc_pool(x, batch_idx):
    mesh = plsc.VectorSubcoreMesh(core_axis_name="c", subcore_axis_name="s")

    @functools.partial(
        pl.kernel,
        out_type=[
            jax.ShapeDtypeStruct((_NW, _B, _D), jnp.float32),
            jax.ShapeDtypeStruct((_NW, _B, _D), jnp.float32),
            jax.ShapeDtypeStruct((_NW, _B, 16), jnp.float32),
        ],
        mesh=mesh,
        scratch_types=[
            pltpu.VMEM((_C, _D), jnp.float32),
            pltpu.VMEM((_C + 16,), jnp.int32),
            pltpu.VMEM((_B, _D), jnp.float32),
            pltpu.VMEM((_B, _D), jnp.float32),
            pltpu.VMEM((_B, 16), jnp.float32),
        ],
    )
    def k(x_hbm, idx_hbm, pmax_hbm, psum_hbm, pcnt_hbm, xbuf, idbuf, amax, asum, acnt):
        cid = lax.axis_index("c")
        sid = lax.axis_index("s")
        wid = sid * 2 + cid

        neg = jnp.full((16,), -jnp.inf, dtype=jnp.float32)
        zero = jnp.zeros((16,), dtype=jnp.float32)
        one = jnp.ones((16,), dtype=jnp.float32)

        def init_body(i, carry):
            b = i // _H
            h = i % _H
            amax[b, pl.ds(h * 16, 16)] = neg
            asum[b, pl.ds(h * 16, 16)] = zero
            return carry

        lax.fori_loop(0, _B * _H, init_body, 0)

        def init_cnt(b, carry):
            acnt[b, :] = zero
            return carry

        lax.fori_loop(0, _B, init_cnt, 0)

        nch = jnp.where(wid < _EXTRA, _CHUNKS_LO + 1, _CHUNKS_LO)
        start = wid * _CHUNKS_LO + jnp.minimum(wid, _EXTRA)

        def chunk_body(j, carry):
            c = start + j
            pltpu.sync_copy(x_hbm.at[pl.ds(c * _C, _C), :], xbuf)
            pltpu.sync_copy(idx_hbm.at[pl.ds(c * _C, _C)], idbuf.at[pl.ds(0, _C)])

            def group_body(g, gcarry):
                idvec = idbuf[pl.ds(g * 16, 16)]
                seg0 = idvec[0]
                seg15 = idvec[15]
                base = g * 16

                @pl.when(seg0 == seg15)
                def _fast():
                    acnt[seg0, :] = acnt[seg0, :] + 16.0

                    def col_body(h, hcarry):
                        sl = pl.ds(h * 16, 16)
                        vs = [xbuf[base + l, sl] for l in range(16)]
                        amax[seg0, sl] = jnp.maximum(amax[seg0, sl], _tree_max(vs))
                        asum[seg0, sl] = asum[seg0, sl] + _tree_sum(vs)
                        return hcarry

                    lax.fori_loop(0, _H, col_body, 0)

                @pl.when(seg0 != seg15)
                def _slow():
                    def lane_body(l, lcarry):
                        r = base + l
                        seg = idbuf[pl.ds(r, 16)][0]
                        acnt[seg, :] = acnt[seg, :] + one

                        def col_body(h, hcarry):
                            sl = pl.ds(h * 16, 16)
                            v = xbuf[r, sl]
                            amax[seg, sl] = jnp.maximum(amax[seg, sl], v)
                            asum[seg, sl] = asum[seg, sl] + v
                            return hcarry

                        lax.fori_loop(0, _H, col_body, 0)
                        return lcarry

                    lax.fori_loop(0, 16, lane_body, 0)

                return gcarry

            lax.fori_loop(0, _G, group_body, 0)
            return carry

        lax.fori_loop(0, nch, chunk_body, 0)

        pltpu.sync_copy(amax, pmax_hbm.at[wid])
        pltpu.sync_copy(asum, psum_hbm.at[wid])
        pltpu.sync_copy(acnt, pcnt_hbm.at[wid])

    return k(x, batch_idx)


def _tc_head_body(pmax_ref, psum_ref, pcnt_ref,
                  w1_ref, b1_ref, g1_ref, be1_ref,
                  w2_ref, b2_ref, g2_ref, be2_ref,
                  w3_ref, b3_ref, out_ref):
    xmax = jnp.max(pmax_ref[...], axis=0)
    xsum = jnp.sum(psum_ref[...], axis=0)
    cnt = jnp.sum(pcnt_ref[...], axis=0)[:, :1]
    xmean = xsum / jnp.maximum(cnt, 1.0)
    xx = jnp.concatenate([xmax, xmean], axis=1)

    def bn_lrelu(h, g, b):
        m = jnp.mean(h, axis=0)
        v = jnp.mean((h - m) ** 2, axis=0)
        h = (h - m) * lax.rsqrt(v + 1e-5) * g + b
        return jnp.where(h > 0, h, 0.01 * h)

    h = jnp.dot(xx, w1_ref[...], preferred_element_type=jnp.float32) + b1_ref[...]
    h = bn_lrelu(h, g1_ref[...], be1_ref[...])
    h = jnp.dot(h, w2_ref[...], preferred_element_type=jnp.float32) + b2_ref[...]
    h = bn_lrelu(h, g2_ref[...], be2_ref[...])
    out_ref[...] = jnp.dot(h, w3_ref[...], preferred_element_type=jnp.float32) + b3_ref[...]


def kernel(x, batch_idx, W1, b1, g1, be1, W2, b2, g2, be2, W3, b3):
    pmax, psum, pcnt = _sc_pool(x, batch_idx)
    out = pl.pallas_call(
        _tc_head_body,
        out_shape=jax.ShapeDtypeStruct((_B, 8), jnp.float32),
    )(pmax, psum, pcnt, W1, b1, g1, be1, W2, b2, g2, be2, W3, b3)
    return out
